# Initial kernel scaffold; baseline (speedup 1.0000x reference)
import jax
import jax.numpy as jnp
from jax.experimental import pallas as pl


def kernel(xh_atoms, xh_residues, xh_intersh, xh_intershp, t, mask_atoms, mask_residues, mask_intersh, mask_intershp, params):
    raise NotImplementedError("write your pallas kernel here")


# speedup vs baseline: 76.4568x; 76.4568x over previous
import functools

import jax
import jax.numpy as jnp
from jax.experimental import pallas as pl
from jax.experimental.pallas import tpu as pltpu

N_DIMS = 3
ATOM_NF = 16
RESIDUE_NF = 21
JOINT_NF = 16
HIDDEN_NF = 64
N_LAYERS = 4
NORM_FACTOR = 100.0
N_BATCH = 16
N_ATOMS = 320
N_RES = 1600
N_NODES = N_ATOMS + N_RES
TILE = 128
NT = N_NODES // TILE


def _silu(x):
    return 0.5 * x * (1.0 + jnp.tanh(0.5 * x))


def _pre_kernel(xa_ref, xr_ref, t_ref,
                wa1, ba1, wa2, ba2,
                wr1, br1, wr2, br2,
                we, be,
                out_ref):
    xa = xa_ref[...]
    xr = xr_ref[...]
    ha = xa[:, N_DIMS:]
    hr = xr[:, N_DIMS:]
    ha = _silu(ha @ wa1[...] + ba1[...]) @ wa2[...] + ba2[...]
    hr = _silu(hr @ wr1[...] + br1[...]) @ wr2[...] + br2[...]
    we_full = we[...]
    we_x = we_full[:N_DIMS, :]
    we_h = we_full[N_DIMS:N_DIMS + JOINT_NF, :]
    we_t = we_full[N_DIMS + JOINT_NF:, :]
    tval = t_ref[...]
    tcontrib = tval * we_t
    fa = xa[:, :N_DIMS] @ we_x + ha @ we_h + tcontrib + be[...]
    fr = xr[:, :N_DIMS] @ we_x + hr @ we_h + tcontrib + be[...]
    out_ref[:N_ATOMS, :] = fa
    out_ref[N_ATOMS:, :] = fr


def _gcl_kernel(ov_ref,
                fi_ref, fj_ref,
                mi_ref, mj_ref,
                w1a, w1b, b1, w2, b2,
                w3f, w3a, b3, w4, b4,
                out_ref,
                acc_ref):
    i = pl.program_id(0)
    j = pl.program_id(1)

    @pl.when(j == 0)
    def _init():
        acc_ref[...] = jnp.zeros_like(acc_ref)

    @pl.when(ov_ref[i, j] > 0)
    def _compute():
        fi = fi_ref[...]
        fj = fj_ref[...]
        a = fi @ w1a[...]
        b = fj @ w1b[...]
        u = a[:, None, :] + b[None, :, :] + b1[...][None, :, :]
        u = _silu(u)
        m = u.reshape(TILE * TILE, HIDDEN_NF) @ w2[...] + b2[...]
        m = _silu(m).reshape(TILE, TILE, HIDDEN_NF)
        mi = mi_ref[...].reshape(TILE, 1)
        mj = mj_ref[...].reshape(1, TILE)
        adj = (mi == mj).astype(jnp.float32)
        contrib = jnp.sum(m * adj[:, :, None], axis=1)
        acc_ref[...] += contrib

    @pl.when(j == NT - 1)
    def _node():
        fi = fi_ref[...]
        agg = acc_ref[...] * (1.0 / NORM_FACTOR)
        tmp = _silu(fi @ w3f[...] + agg @ w3a[...] + b3[...])
        out_ref[...] = fi + tmp @ w4[...] + b4[...]


def _post_kernel(feat_ref, maskf_ref,
                 weo, beo,
                 wad1, bad1, wad2, bad2,
                 wrd1, brd1, wrd2, brd2,
                 outa_ref, outr_ref):
    feat = feat_ref[...]
    out = feat @ weo[...] + beo[...]
    vel = out[:, :N_DIMS]
    hfin = out[:, N_DIMS:N_DIMS + JOINT_NF]
    ha = _silu(hfin[:N_ATOMS] @ wad1[...] + bad1[...]) @ wad2[...] + bad2[...]
    hr = _silu(hfin[N_ATOMS:] @ wrd1[...] + brd1[...]) @ wrd2[...] + brd2[...]
    maski = maskf_ref[...]
    batches = jax.lax.broadcasted_iota(jnp.int32, (N_NODES, N_BATCH), 1)
    onehot = (maski == batches).astype(jnp.float32)
    seg = jax.lax.dot_general(onehot, vel, (((0,), (0,)), ((), ())))
    cnt = jnp.sum(onehot, axis=0, keepdims=True).T
    mean = seg / jnp.maximum(cnt, 1.0)
    vel = vel - onehot @ mean
    outa_ref[...] = jnp.concatenate([vel[:N_ATOMS], ha], axis=1)
    outr_ref[...] = jnp.concatenate([vel[N_ATOMS:], hr], axis=1)


@jax.jit
def _run(xh_atoms, xh_residues, t, mask_atoms, mask_residues, params):
    mask = jnp.concatenate([mask_atoms, mask_residues]).astype(jnp.int32)
    maskf = mask.astype(jnp.float32)

    mt = mask.reshape(NT, TILE)
    tmin = jnp.min(mt, axis=1)
    tmax = jnp.max(mt, axis=1)
    ov = ((tmin[:, None] <= tmax[None, :]) &
          (tmin[None, :] <= tmax[:, None])).astype(jnp.int32)

    p = params

    def lin(name, sub=None):
        q = p[name] if sub is None else p[name][sub]
        return q["w"], q["b"].reshape(1, -1)

    wa1, ba1 = lin("atom_encoder", "l1")
    wa2, ba2 = lin("atom_encoder", "l2")
    wr1, br1 = lin("residue_encoder", "l1")
    wr2, br2 = lin("residue_encoder", "l2")
    we, be = p["gnn"]["embedding"]["w"], p["gnn"]["embedding"]["b"].reshape(1, -1)

    feat = pl.pallas_call(
        _pre_kernel,
        out_shape=jax.ShapeDtypeStruct((N_NODES, HIDDEN_NF), jnp.float32),
    )(xh_atoms, xh_residues, t.reshape(1, 1),
      wa1, ba1, wa2, ba2, wr1, br1, wr2, br2, we, be)

    mask_col = maskf.reshape(NT, TILE, 1)
    mask_row = maskf.reshape(NT, 1, TILE)

    def wspec(shape):
        nd = len(shape)
        return pl.BlockSpec(shape, lambda i, j: (0,) * nd)

    for layer in p["gnn"]["gcl"]:
        w1 = layer["edge_mlp"]["l1"]["w"]
        w1a = w1[:HIDDEN_NF, :]
        w1b = w1[HIDDEN_NF:, :]
        b1 = layer["edge_mlp"]["l1"]["b"].reshape(1, HIDDEN_NF)
        w2 = layer["edge_mlp"]["l2"]["w"]
        b2 = layer["edge_mlp"]["l2"]["b"].reshape(1, HIDDEN_NF)
        w3 = layer["node_mlp"]["l1"]["w"]
        w3f = w3[:HIDDEN_NF, :]
        w3a = w3[HIDDEN_NF:, :]
        b3 = layer["node_mlp"]["l1"]["b"].reshape(1, HIDDEN_NF)
        w4 = layer["node_mlp"]["l2"]["w"]
        b4 = layer["node_mlp"]["l2"]["b"].reshape(1, HIDDEN_NF)

        weights = (w1a, w1b, b1, w2, b2, w3f, w3a, b3, w4, b4)
        feat = pl.pallas_call(
            _gcl_kernel,
            grid=(NT, NT),
            in_specs=[
                pl.BlockSpec(memory_space=pltpu.SMEM),
                pl.BlockSpec((TILE, HIDDEN_NF), lambda i, j: (i, 0)),
                pl.BlockSpec((TILE, HIDDEN_NF), lambda i, j: (j, 0)),
                pl.BlockSpec((1, TILE, 1), lambda i, j: (i, 0, 0)),
                pl.BlockSpec((1, 1, TILE), lambda i, j: (j, 0, 0)),
            ] + [wspec(w.shape) for w in weights],
            out_specs=pl.BlockSpec((TILE, HIDDEN_NF), lambda i, j: (i, 0)),
            out_shape=jax.ShapeDtypeStruct((N_NODES, HIDDEN_NF), jnp.float32),
            scratch_shapes=[pltpu.VMEM((TILE, HIDDEN_NF), jnp.float32)],
        )(ov, feat, feat, mask_col, mask_row, *weights)

    weo, beo = p["gnn"]["embedding_out"]["w"], p["gnn"]["embedding_out"]["b"].reshape(1, -1)
    wad1, bad1 = lin("atom_decoder", "l1")
    wad2, bad2 = lin("atom_decoder", "l2")
    wrd1, brd1 = lin("residue_decoder", "l1")
    wrd2, brd2 = lin("residue_decoder", "l2")

    outa, outr = pl.pallas_call(
        _post_kernel,
        out_shape=(jax.ShapeDtypeStruct((N_ATOMS, N_DIMS + ATOM_NF), jnp.float32),
                   jax.ShapeDtypeStruct((N_RES, N_DIMS + RESIDUE_NF), jnp.float32)),
    )(feat, mask.reshape(N_NODES, 1),
      weo, beo, wad1, bad1, wad2, bad2, wrd1, brd1, wrd2, brd2)

    return outa, outr


def kernel(xh_atoms, xh_residues, xh_intersh, xh_intershp, t,
           mask_atoms, mask_residues, mask_intersh, mask_intershp, params):
    return _run(xh_atoms, xh_residues, t, mask_atoms, mask_residues, params)


# speedup vs baseline: 176.6367x; 2.3103x over previous
import jax
import jax.numpy as jnp
from jax.experimental import pallas as pl
from jax.experimental.pallas import tpu as pltpu

N_DIMS = 3
ATOM_NF = 16
RESIDUE_NF = 21
JOINT_NF = 16
HIDDEN_NF = 64
N_LAYERS = 4
NORM_FACTOR = 100.0
N_BATCH = 16
N_ATOMS = 320
N_RES = 1600
N_NODES = N_ATOMS + N_RES
TILE = 128
NT = N_NODES // TILE


def _silu(x):
    return 0.5 * x * (1.0 + jnp.tanh(0.5 * x))


def _perm_matrix(permcol):
    iota = jax.lax.broadcasted_iota(jnp.int32, (N_NODES, N_NODES), 1)
    return (permcol == iota).astype(jnp.float32)


def _pre_kernel(xa_ref, xr_ref, t_ref, perm_ref,
                wa1, ba1, wa2, ba2,
                wr1, br1, wr2, br2,
                we, be,
                out_ref):
    xa = xa_ref[...]
    xr = xr_ref[...]
    ha = xa[:, N_DIMS:]
    hr = xr[:, N_DIMS:]
    ha = _silu(ha @ wa1[...] + ba1[...]) @ wa2[...] + ba2[...]
    hr = _silu(hr @ wr1[...] + br1[...]) @ wr2[...] + br2[...]
    we_full = we[...]
    we_x = we_full[:N_DIMS, :]
    we_h = we_full[N_DIMS:N_DIMS + JOINT_NF, :]
    we_t = we_full[N_DIMS + JOINT_NF:, :]
    tval = t_ref[...]
    tcontrib = tval * we_t
    fa = xa[:, :N_DIMS] @ we_x + ha @ we_h + tcontrib + be[...]
    fr = xr[:, :N_DIMS] @ we_x + hr @ we_h + tcontrib + be[...]
    feat = jnp.concatenate([fa, fr], axis=0)
    out_ref[...] = _perm_matrix(perm_ref[...]) @ feat


def _gcl_kernel(bounds_ref,
                feat0_ref,
                mcol_ref,
                mrow_ref,
                w1a, w1b, b1, w2, b2,
                w3f, w3a, b3, w4, b4,
                out_ref,
                fbuf):
    l = pl.program_id(0)
    i = pl.program_id(1)

    @pl.when((l == 0) & (i == 0))
    def _copy_in():
        fbuf[0] = feat0_ref[...]

    cur = jax.lax.rem(l, 2)
    nxt = 1 - cur
    fi = fbuf[cur, pl.ds(i * TILE, TILE), :]
    a = fi @ w1a[0] + b1[0]
    mi = mcol_ref[...]
    jlo = bounds_ref[0, i]
    jhi = bounds_ref[1, i]

    def body(j, acc):
        fj = fbuf[cur, pl.ds(j * TILE, TILE), :]
        b = fj @ w1b[0]
        u = _silu(a[:, None, :] + b[None, :, :])
        m = _silu(u.reshape(TILE * TILE, HIDDEN_NF) @ w2[0] + b2[0])
        m = m.reshape(TILE, TILE, HIDDEN_NF)
        mj = mrow_ref[:, pl.ds(j * TILE, TILE)]
        adj = (mi == mj).astype(jnp.float32)
        return acc + jnp.sum(m * adj[:, :, None], axis=1)

    acc = jax.lax.fori_loop(jlo, jhi + 1, body,
                            jnp.zeros((TILE, HIDDEN_NF), jnp.float32))
    agg = acc * (1.0 / NORM_FACTOR)
    tmp = _silu(fi @ w3f[0] + agg @ w3a[0] + b3[0])
    newf = fi + tmp @ w4[0] + b4[0]
    fbuf[nxt, pl.ds(i * TILE, TILE), :] = newf

    @pl.when(l == N_LAYERS - 1)
    def _emit():
        out_ref[pl.ds(i * TILE, TILE), :] = newf


def _post_kernel(featp_ref, perm_ref, maski_ref,
                 weo, beo,
                 wad1, bad1, wad2, bad2,
                 wrd1, brd1, wrd2, brd2,
                 outa_ref, outr_ref):
    pmat = _perm_matrix(perm_ref[...])
    feat = jax.lax.dot_general(pmat, featp_ref[...], (((0,), (0,)), ((), ())))
    out = feat @ weo[...] + beo[...]
    vel = out[:, :N_DIMS]
    hfin = out[:, N_DIMS:N_DIMS + JOINT_NF]
    ha = _silu(hfin[:N_ATOMS] @ wad1[...] + bad1[...]) @ wad2[...] + bad2[...]
    hr = _silu(hfin[N_ATOMS:] @ wrd1[...] + brd1[...]) @ wrd2[...] + brd2[...]
    maski = maski_ref[...]
    batches = jax.lax.broadcasted_iota(jnp.int32, (N_NODES, N_BATCH), 1)
    onehot = (maski == batches).astype(jnp.float32)
    seg = jax.lax.dot_general(onehot, vel, (((0,), (0,)), ((), ())))
    cnt = jnp.sum(onehot, axis=0, keepdims=True).T
    mean = seg / jnp.maximum(cnt, 1.0)
    vel = vel - onehot @ mean
    outa_ref[...] = jnp.concatenate([vel[:N_ATOMS], ha], axis=1)
    outr_ref[...] = jnp.concatenate([vel[N_ATOMS:], hr], axis=1)


@jax.jit
def _run(xh_atoms, xh_residues, t, mask_atoms, mask_residues, params):
    mask = jnp.concatenate([mask_atoms, mask_residues]).astype(jnp.int32)
    perm = jnp.argsort(mask, stable=True).astype(jnp.int32)
    maskp = mask[perm]
    maskpf = maskp.astype(jnp.float32)

    mt = maskp.reshape(NT, TILE)
    vmin = mt[:, 0]
    vmax = mt[:, -1]
    jlo = (jnp.searchsorted(maskp, vmin, side="left") // TILE).astype(jnp.int32)
    jhi = ((jnp.searchsorted(maskp, vmax, side="right") - 1) // TILE).astype(jnp.int32)
    bounds = jnp.stack([jlo, jhi])

    p = params

    def lin(name, sub=None):
        q = p[name] if sub is None else p[name][sub]
        return q["w"], q["b"].reshape(1, -1)

    wa1, ba1 = lin("atom_encoder", "l1")
    wa2, ba2 = lin("atom_encoder", "l2")
    wr1, br1 = lin("residue_encoder", "l1")
    wr2, br2 = lin("residue_encoder", "l2")
    we, be = p["gnn"]["embedding"]["w"], p["gnn"]["embedding"]["b"].reshape(1, -1)

    featp = pl.pallas_call(
        _pre_kernel,
        out_shape=jax.ShapeDtypeStruct((N_NODES, HIDDEN_NF), jnp.float32),
    )(xh_atoms, xh_residues, t.reshape(1, 1), perm.reshape(N_NODES, 1),
      wa1, ba1, wa2, ba2, wr1, br1, wr2, br2, we, be)

    gcl = p["gnn"]["gcl"]

    def stack(path):
        return jnp.stack([path(layer) for layer in gcl])

    w1a_s = stack(lambda q: q["edge_mlp"]["l1"]["w"][:HIDDEN_NF, :])
    w1b_s = stack(lambda q: q["edge_mlp"]["l1"]["w"][HIDDEN_NF:, :])
    b1_s = stack(lambda q: q["edge_mlp"]["l1"]["b"].reshape(1, HIDDEN_NF))
    w2_s = stack(lambda q: q["edge_mlp"]["l2"]["w"])
    b2_s = stack(lambda q: q["edge_mlp"]["l2"]["b"].reshape(1, HIDDEN_NF))
    w3f_s = stack(lambda q: q["node_mlp"]["l1"]["w"][:HIDDEN_NF, :])
    w3a_s = stack(lambda q: q["node_mlp"]["l1"]["w"][HIDDEN_NF:, :])
    b3_s = stack(lambda q: q["node_mlp"]["l1"]["b"].reshape(1, HIDDEN_NF))
    w4_s = stack(lambda q: q["node_mlp"]["l2"]["w"])
    b4_s = stack(lambda q: q["node_mlp"]["l2"]["b"].reshape(1, HIDDEN_NF))

    def wspec(shape):
        nd = len(shape) - 1
        return pl.BlockSpec((1,) + shape[1:], lambda l, i: (l,) + (0,) * nd)

    featp = pl.pallas_call(
        _gcl_kernel,
        grid=(N_LAYERS, NT),
        in_specs=[
            pl.BlockSpec(memory_space=pltpu.SMEM),
            pl.BlockSpec((N_NODES, HIDDEN_NF), lambda l, i: (0, 0)),
            pl.BlockSpec((TILE, 1), lambda l, i: (i, 0)),
            pl.BlockSpec((1, N_NODES), lambda l, i: (0, 0)),
            wspec(w1a_s.shape), wspec(w1b_s.shape), wspec(b1_s.shape),
            wspec(w2_s.shape), wspec(b2_s.shape),
            wspec(w3f_s.shape), wspec(w3a_s.shape), wspec(b3_s.shape),
            wspec(w4_s.shape), wspec(b4_s.shape),
        ],
        out_specs=pl.BlockSpec((N_NODES, HIDDEN_NF), lambda l, i: (0, 0)),
        out_shape=jax.ShapeDtypeStruct((N_NODES, HIDDEN_NF), jnp.float32),
        scratch_shapes=[pltpu.VMEM((2, N_NODES, HIDDEN_NF), jnp.float32)],
    )(bounds, featp, maskpf.reshape(N_NODES, 1), maskpf.reshape(1, N_NODES),
      w1a_s, w1b_s, b1_s, w2_s, b2_s, w3f_s, w3a_s, b3_s, w4_s, b4_s)

    weo, beo = p["gnn"]["embedding_out"]["w"], p["gnn"]["embedding_out"]["b"].reshape(1, -1)
    wad1, bad1 = lin("atom_decoder", "l1")
    wad2, bad2 = lin("atom_decoder", "l2")
    wrd1, brd1 = lin("residue_decoder", "l1")
    wrd2, brd2 = lin("residue_decoder", "l2")

    outa, outr = pl.pallas_call(
        _post_kernel,
        out_shape=(jax.ShapeDtypeStruct((N_ATOMS, N_DIMS + ATOM_NF), jnp.float32),
                   jax.ShapeDtypeStruct((N_RES, N_DIMS + RESIDUE_NF), jnp.float32)),
    )(featp, perm.reshape(N_NODES, 1), mask.reshape(N_NODES, 1),
      weo, beo, wad1, bad1, wad2, bad2, wrd1, brd1, wrd2, brd2)

    return outa, outr


def kernel(xh_atoms, xh_residues, xh_intersh, xh_intershp, t,
           mask_atoms, mask_residues, mask_intersh, mask_intershp, params):
    return _run(xh_atoms, xh_residues, t, mask_atoms, mask_residues, params)


# speedup vs baseline: 222.2884x; 1.2585x over previous
import jax
import jax.numpy as jnp
from jax.experimental import pallas as pl
from jax.experimental.pallas import tpu as pltpu

N_DIMS = 3
ATOM_NF = 16
RESIDUE_NF = 21
JOINT_NF = 16
HIDDEN_NF = 64
N_LAYERS = 4
NORM_FACTOR = 100.0
N_BATCH = 16
N_ATOMS = 320
N_RES = 1600
N_NODES = N_ATOMS + N_RES
TILE = 64
NT = N_NODES // TILE


def _silu(x):
    return 0.5 * x * (1.0 + jnp.tanh(0.5 * x))


def _perm_matrix(permcol):
    iota = jax.lax.broadcasted_iota(jnp.int32, (N_NODES, N_NODES), 1)
    return (permcol == iota).astype(jnp.float32)


def _pre_kernel(xa_ref, xr_ref, t_ref, perm_ref,
                wa1, ba1, wa2, ba2,
                wr1, br1, wr2, br2,
                we, be,
                out_ref):
    xa = xa_ref[...]
    xr = xr_ref[...]
    ha = xa[:, N_DIMS:]
    hr = xr[:, N_DIMS:]
    ha = _silu(ha @ wa1[...] + ba1[...]) @ wa2[...] + ba2[...]
    hr = _silu(hr @ wr1[...] + br1[...]) @ wr2[...] + br2[...]
    we_full = we[...]
    we_x = we_full[:N_DIMS, :]
    we_h = we_full[N_DIMS:N_DIMS + JOINT_NF, :]
    we_t = we_full[N_DIMS + JOINT_NF:, :]
    tval = t_ref[...]
    tcontrib = tval * we_t
    fa = xa[:, :N_DIMS] @ we_x + ha @ we_h + tcontrib + be[...]
    fr = xr[:, :N_DIMS] @ we_x + hr @ we_h + tcontrib + be[...]
    feat = jnp.concatenate([fa, fr], axis=0)
    out_ref[...] = _perm_matrix(perm_ref[...]) @ feat


def _gcl_kernel(bounds_ref,
                feat0_ref,
                mcol_ref,
                mrow_ref,
                w1a, w1b, b1, w2, b2,
                w3f, w3a, b3, w4, b4,
                out_ref,
                fbuf):
    l = pl.program_id(0)
    i = pl.program_id(1)

    @pl.when((l == 0) & (i == 0))
    def _copy_in():
        fbuf[0] = feat0_ref[...]

    cur = jax.lax.rem(l, 2)
    nxt = 1 - cur
    fi = fbuf[cur, pl.ds(i * TILE, TILE), :]
    a = fi @ w1a[0] + b1[0]
    mi = mcol_ref[...]
    jlo = bounds_ref[0, i]
    jhi = bounds_ref[1, i]

    def body(j, acc):
        fj = fbuf[cur, pl.ds(pl.multiple_of(j * TILE, TILE), TILE), :]
        b = fj @ w1b[0]
        u = _silu(a[:, None, :] + b[None, :, :])
        m = _silu(u.reshape(TILE * TILE, HIDDEN_NF) @ w2[0] + b2[0])
        m = m.reshape(TILE, TILE, HIDDEN_NF)
        mj = mrow_ref[j]
        adj = (mi == mj).astype(jnp.float32)
        return acc + jnp.sum(m * adj[:, :, None], axis=1)

    acc = jax.lax.fori_loop(jlo, jhi + 1, body,
                            jnp.zeros((TILE, HIDDEN_NF), jnp.float32))
    agg = acc * (1.0 / NORM_FACTOR)
    tmp = _silu(fi @ w3f[0] + agg @ w3a[0] + b3[0])
    newf = fi + tmp @ w4[0] + b4[0]
    fbuf[nxt, pl.ds(i * TILE, TILE), :] = newf

    @pl.when(l == N_LAYERS - 1)
    def _emit():
        out_ref[pl.ds(i * TILE, TILE), :] = newf


def _post_kernel(featp_ref, perm_ref, maski_ref,
                 weo, beo,
                 wad1, bad1, wad2, bad2,
                 wrd1, brd1, wrd2, brd2,
                 outa_ref, outr_ref):
    pmat = _perm_matrix(perm_ref[...])
    feat = jax.lax.dot_general(pmat, featp_ref[...], (((0,), (0,)), ((), ())))
    out = feat @ weo[...] + beo[...]
    vel = out[:, :N_DIMS]
    hfin = out[:, N_DIMS:N_DIMS + JOINT_NF]
    ha = _silu(hfin[:N_ATOMS] @ wad1[...] + bad1[...]) @ wad2[...] + bad2[...]
    hr = _silu(hfin[N_ATOMS:] @ wrd1[...] + brd1[...]) @ wrd2[...] + brd2[...]
    maski = maski_ref[...]
    batches = jax.lax.broadcasted_iota(jnp.int32, (N_NODES, N_BATCH), 1)
    onehot = (maski == batches).astype(jnp.float32)
    seg = jax.lax.dot_general(onehot, vel, (((0,), (0,)), ((), ())))
    cnt = jnp.sum(onehot, axis=0, keepdims=True).T
    mean = seg / jnp.maximum(cnt, 1.0)
    vel = vel - onehot @ mean
    outa_ref[...] = jnp.concatenate([vel[:N_ATOMS], ha], axis=1)
    outr_ref[...] = jnp.concatenate([vel[N_ATOMS:], hr], axis=1)


@jax.jit
def _run(xh_atoms, xh_residues, t, mask_atoms, mask_residues, params):
    mask = jnp.concatenate([mask_atoms, mask_residues]).astype(jnp.int32)
    perm = jnp.argsort(mask, stable=True).astype(jnp.int32)
    maskp = mask[perm]
    maskpf = maskp.astype(jnp.float32)

    mt = maskp.reshape(NT, TILE)
    vmin = mt[:, 0]
    vmax = mt[:, -1]
    jlo = (jnp.searchsorted(maskp, vmin, side="left") // TILE).astype(jnp.int32)
    jhi = ((jnp.searchsorted(maskp, vmax, side="right") - 1) // TILE).astype(jnp.int32)
    bounds = jnp.stack([jlo, jhi])

    p = params

    def lin(name, sub=None):
        q = p[name] if sub is None else p[name][sub]
        return q["w"], q["b"].reshape(1, -1)

    wa1, ba1 = lin("atom_encoder", "l1")
    wa2, ba2 = lin("atom_encoder", "l2")
    wr1, br1 = lin("residue_encoder", "l1")
    wr2, br2 = lin("residue_encoder", "l2")
    we, be = p["gnn"]["embedding"]["w"], p["gnn"]["embedding"]["b"].reshape(1, -1)

    featp = pl.pallas_call(
        _pre_kernel,
        out_shape=jax.ShapeDtypeStruct((N_NODES, HIDDEN_NF), jnp.float32),
    )(xh_atoms, xh_residues, t.reshape(1, 1), perm.reshape(N_NODES, 1),
      wa1, ba1, wa2, ba2, wr1, br1, wr2, br2, we, be)

    gcl = p["gnn"]["gcl"]

    def stack(path):
        return jnp.stack([path(layer) for layer in gcl])

    w1a_s = stack(lambda q: q["edge_mlp"]["l1"]["w"][:HIDDEN_NF, :])
    w1b_s = stack(lambda q: q["edge_mlp"]["l1"]["w"][HIDDEN_NF:, :])
    b1_s = stack(lambda q: q["edge_mlp"]["l1"]["b"].reshape(1, HIDDEN_NF))
    w2_s = stack(lambda q: q["edge_mlp"]["l2"]["w"])
    b2_s = stack(lambda q: q["edge_mlp"]["l2"]["b"].reshape(1, HIDDEN_NF))
    w3f_s = stack(lambda q: q["node_mlp"]["l1"]["w"][:HIDDEN_NF, :])
    w3a_s = stack(lambda q: q["node_mlp"]["l1"]["w"][HIDDEN_NF:, :])
    b3_s = stack(lambda q: q["node_mlp"]["l1"]["b"].reshape(1, HIDDEN_NF))
    w4_s = stack(lambda q: q["node_mlp"]["l2"]["w"])
    b4_s = stack(lambda q: q["node_mlp"]["l2"]["b"].reshape(1, HIDDEN_NF))

    def wspec(shape):
        nd = len(shape) - 1
        return pl.BlockSpec((1,) + shape[1:], lambda l, i: (l,) + (0,) * nd)

    featp = pl.pallas_call(
        _gcl_kernel,
        grid=(N_LAYERS, NT),
        in_specs=[
            pl.BlockSpec(memory_space=pltpu.SMEM),
            pl.BlockSpec((N_NODES, HIDDEN_NF), lambda l, i: (0, 0)),
            pl.BlockSpec((TILE, 1), lambda l, i: (i, 0)),
            pl.BlockSpec((NT, 1, TILE), lambda l, i: (0, 0, 0)),
            wspec(w1a_s.shape), wspec(w1b_s.shape), wspec(b1_s.shape),
            wspec(w2_s.shape), wspec(b2_s.shape),
            wspec(w3f_s.shape), wspec(w3a_s.shape), wspec(b3_s.shape),
            wspec(w4_s.shape), wspec(b4_s.shape),
        ],
        out_specs=pl.BlockSpec((N_NODES, HIDDEN_NF), lambda l, i: (0, 0)),
        out_shape=jax.ShapeDtypeStruct((N_NODES, HIDDEN_NF), jnp.float32),
        scratch_shapes=[pltpu.VMEM((2, N_NODES, HIDDEN_NF), jnp.float32)],
    )(bounds, featp, maskpf.reshape(N_NODES, 1), maskpf.reshape(NT, 1, TILE),
      w1a_s, w1b_s, b1_s, w2_s, b2_s, w3f_s, w3a_s, b3_s, w4_s, b4_s)

    weo, beo = p["gnn"]["embedding_out"]["w"], p["gnn"]["embedding_out"]["b"].reshape(1, -1)
    wad1, bad1 = lin("atom_decoder", "l1")
    wad2, bad2 = lin("atom_decoder", "l2")
    wrd1, brd1 = lin("residue_decoder", "l1")
    wrd2, brd2 = lin("residue_decoder", "l2")

    outa, outr = pl.pallas_call(
        _post_kernel,
        out_shape=(jax.ShapeDtypeStruct((N_ATOMS, N_DIMS + ATOM_NF), jnp.float32),
                   jax.ShapeDtypeStruct((N_RES, N_DIMS + RESIDUE_NF), jnp.float32)),
    )(featp, perm.reshape(N_NODES, 1), mask.reshape(N_NODES, 1),
      weo, beo, wad1, bad1, wad2, bad2, wrd1, brd1, wrd2, brd2)

    return outa, outr


def kernel(xh_atoms, xh_residues, xh_intersh, xh_intershp, t,
           mask_atoms, mask_residues, mask_intersh, mask_intershp, params):
    return _run(xh_atoms, xh_residues, t, mask_atoms, mask_residues, params)


# speedup vs baseline: 270.1507x; 1.2153x over previous
import jax
import jax.numpy as jnp
from jax.experimental import pallas as pl
from jax.experimental.pallas import tpu as pltpu

N_DIMS = 3
ATOM_NF = 16
RESIDUE_NF = 21
JOINT_NF = 16
HIDDEN_NF = 64
N_LAYERS = 4
NORM_FACTOR = 100.0
N_BATCH = 16
N_ATOMS = 320
N_RES = 1600
N_NODES = N_ATOMS + N_RES
TILE = 64
NT = N_NODES // TILE


def _silu(x):
    return 0.5 * x * (1.0 + jnp.tanh(0.5 * x))


def _perm_matrix(permcol):
    iota = jax.lax.broadcasted_iota(jnp.int32, (N_NODES, N_NODES), 1)
    return (permcol == iota).astype(jnp.float32)


def _pre_kernel(xa_ref, xr_ref, t_ref, perm_ref,
                wa1, ba1, wa2, ba2,
                wr1, br1, wr2, br2,
                we, be,
                out_ref):
    xa = xa_ref[...]
    xr = xr_ref[...]
    ha = xa[:, N_DIMS:]
    hr = xr[:, N_DIMS:]
    ha = _silu(ha @ wa1[...] + ba1[...]) @ wa2[...] + ba2[...]
    hr = _silu(hr @ wr1[...] + br1[...]) @ wr2[...] + br2[...]
    we_full = we[...]
    we_x = we_full[:N_DIMS, :]
    we_h = we_full[N_DIMS:N_DIMS + JOINT_NF, :]
    we_t = we_full[N_DIMS + JOINT_NF:, :]
    tval = t_ref[...]
    tcontrib = tval * we_t
    fa = xa[:, :N_DIMS] @ we_x + ha @ we_h + tcontrib + be[...]
    fr = xr[:, :N_DIMS] @ we_x + hr @ we_h + tcontrib + be[...]
    feat = jnp.concatenate([fa, fr], axis=0)
    out_ref[...] = _perm_matrix(perm_ref[...]) @ feat


TJ = 128
TJH = TJ // 2
NTJ = N_NODES // TJ
FOLD = 2 * HIDDEN_NF


def _gcl_kernel(bounds_ref,
                feat0_ref,
                mcol_ref,
                mjf_ref,
                w1a, b1, w1b, w2d, b2d,
                w3f, w3a, b3, w4, b4,
                out_ref,
                fbuf):
    l = pl.program_id(0)
    i = pl.program_id(1)

    @pl.when((l == 0) & (i == 0))
    def _copy_in():
        fbuf[0] = feat0_ref[...]

    cur = jax.lax.rem(l, 2)
    nxt = 1 - cur
    fi = fbuf[cur, pl.ds(i * TILE, TILE), :]
    a = fi @ w1a[0] + b1[0]
    a2 = jnp.concatenate([a, a], axis=1)
    mi3 = mcol_ref[...].reshape(TILE, 1, 1)
    jlo = bounds_ref[0, i]
    jhi = bounds_ref[1, i]

    def body(j, acc):
        fj = fbuf[cur, pl.ds(j * TJ, TJ), :]
        b = fj @ w1b[0]
        bf = jnp.concatenate([b[:TJH], b[TJH:]], axis=1)
        u = _silu(a2[:, None, :] + bf[None, :, :])
        m = _silu(u.reshape(TILE * TJH, FOLD) @ w2d[0] + b2d[0])
        m = m.reshape(TILE, TJH, FOLD)
        adjf = (mi3 == mjf_ref[j][None, :, :]).astype(jnp.float32)
        s = jnp.sum(m * adjf, axis=1)
        return acc + s[:, :HIDDEN_NF] + s[:, HIDDEN_NF:]

    acc = jax.lax.fori_loop(jlo, jhi + 1, body,
                            jnp.zeros((TILE, HIDDEN_NF), jnp.float32))
    agg = acc * (1.0 / NORM_FACTOR)
    tmp = _silu(fi @ w3f[0] + agg @ w3a[0] + b3[0])
    newf = fi + tmp @ w4[0] + b4[0]
    fbuf[nxt, pl.ds(i * TILE, TILE), :] = newf

    @pl.when(l == N_LAYERS - 1)
    def _emit():
        out_ref[pl.ds(i * TILE, TILE), :] = newf


def _post_kernel(featp_ref, perm_ref, maski_ref,
                 weo, beo,
                 wad1, bad1, wad2, bad2,
                 wrd1, brd1, wrd2, brd2,
                 outa_ref, outr_ref):
    pmat = _perm_matrix(perm_ref[...])
    feat = jax.lax.dot_general(pmat, featp_ref[...], (((0,), (0,)), ((), ())))
    out = feat @ weo[...] + beo[...]
    vel = out[:, :N_DIMS]
    hfin = out[:, N_DIMS:N_DIMS + JOINT_NF]
    ha = _silu(hfin[:N_ATOMS] @ wad1[...] + bad1[...]) @ wad2[...] + bad2[...]
    hr = _silu(hfin[N_ATOMS:] @ wrd1[...] + brd1[...]) @ wrd2[...] + brd2[...]
    maski = maski_ref[...]
    batches = jax.lax.broadcasted_iota(jnp.int32, (N_NODES, N_BATCH), 1)
    onehot = (maski == batches).astype(jnp.float32)
    seg = jax.lax.dot_general(onehot, vel, (((0,), (0,)), ((), ())))
    cnt = jnp.sum(onehot, axis=0, keepdims=True).T
    mean = seg / jnp.maximum(cnt, 1.0)
    vel = vel - onehot @ mean
    outa_ref[...] = jnp.concatenate([vel[:N_ATOMS], ha], axis=1)
    outr_ref[...] = jnp.concatenate([vel[N_ATOMS:], hr], axis=1)


@jax.jit
def _run(xh_atoms, xh_residues, t, mask_atoms, mask_residues, params):
    mask = jnp.concatenate([mask_atoms, mask_residues]).astype(jnp.int32)
    perm = jnp.argsort(mask, stable=True).astype(jnp.int32)
    maskp = mask[perm]
    maskpf = maskp.astype(jnp.float32)

    mt = maskp.reshape(NT, TILE)
    vmin = mt[:, 0]
    vmax = mt[:, -1]
    jlo = (jnp.searchsorted(maskp, vmin, side="left") // TJ).astype(jnp.int32)
    jhi = ((jnp.searchsorted(maskp, vmax, side="right") - 1) // TJ).astype(jnp.int32)
    bounds = jnp.stack([jlo, jhi])

    p = params

    def lin(name, sub=None):
        q = p[name] if sub is None else p[name][sub]
        return q["w"], q["b"].reshape(1, -1)

    wa1, ba1 = lin("atom_encoder", "l1")
    wa2, ba2 = lin("atom_encoder", "l2")
    wr1, br1 = lin("residue_encoder", "l1")
    wr2, br2 = lin("residue_encoder", "l2")
    we, be = p["gnn"]["embedding"]["w"], p["gnn"]["embedding"]["b"].reshape(1, -1)

    featp = pl.pallas_call(
        _pre_kernel,
        out_shape=jax.ShapeDtypeStruct((N_NODES, HIDDEN_NF), jnp.float32),
    )(xh_atoms, xh_residues, t.reshape(1, 1), perm.reshape(N_NODES, 1),
      wa1, ba1, wa2, ba2, wr1, br1, wr2, br2, we, be)

    gcl = p["gnn"]["gcl"]

    def stack(path):
        return jnp.stack([path(layer) for layer in gcl])

    def bdiag(w):
        return jnp.kron(jnp.eye(2, dtype=w.dtype), w)

    w1a_s = stack(lambda q: q["edge_mlp"]["l1"]["w"][:HIDDEN_NF, :])
    w1b_s = stack(lambda q: q["edge_mlp"]["l1"]["w"][HIDDEN_NF:, :])
    b1_s = stack(lambda q: q["edge_mlp"]["l1"]["b"].reshape(1, HIDDEN_NF))
    w2d_s = stack(lambda q: bdiag(q["edge_mlp"]["l2"]["w"]))
    b2d_s = stack(lambda q: jnp.tile(q["edge_mlp"]["l2"]["b"].reshape(1, HIDDEN_NF), (1, 2)))
    w3f_s = stack(lambda q: q["node_mlp"]["l1"]["w"][:HIDDEN_NF, :])
    w3a_s = stack(lambda q: q["node_mlp"]["l1"]["w"][HIDDEN_NF:, :])
    b3_s = stack(lambda q: q["node_mlp"]["l1"]["b"].reshape(1, HIDDEN_NF))
    w4_s = stack(lambda q: q["node_mlp"]["l2"]["w"])
    b4_s = stack(lambda q: q["node_mlp"]["l2"]["b"].reshape(1, HIDDEN_NF))

    def wspec(shape):
        nd = len(shape) - 1
        return pl.BlockSpec((1,) + shape[1:], lambda l, i: (l,) + (0,) * nd)

    mc = maskpf.reshape(NTJ, 2, TJH)
    mjf = jnp.concatenate(
        [jnp.broadcast_to(mc[:, 0, :, None], (NTJ, TJH, HIDDEN_NF)),
         jnp.broadcast_to(mc[:, 1, :, None], (NTJ, TJH, HIDDEN_NF))], axis=2)

    featp = pl.pallas_call(
        _gcl_kernel,
        grid=(N_LAYERS, NT),
        in_specs=[
            pl.BlockSpec(memory_space=pltpu.SMEM),
            pl.BlockSpec((N_NODES, HIDDEN_NF), lambda l, i: (0, 0)),
            pl.BlockSpec((TILE, 1), lambda l, i: (i, 0)),
            pl.BlockSpec((NTJ, TJH, FOLD), lambda l, i: (0, 0, 0)),
            wspec(w1a_s.shape), wspec(b1_s.shape), wspec(w1b_s.shape),
            wspec(w2d_s.shape), wspec(b2d_s.shape),
            wspec(w3f_s.shape), wspec(w3a_s.shape), wspec(b3_s.shape),
            wspec(w4_s.shape), wspec(b4_s.shape),
        ],
        out_specs=pl.BlockSpec((N_NODES, HIDDEN_NF), lambda l, i: (0, 0)),
        out_shape=jax.ShapeDtypeStruct((N_NODES, HIDDEN_NF), jnp.float32),
        scratch_shapes=[pltpu.VMEM((2, N_NODES, HIDDEN_NF), jnp.float32)],
    )(bounds, featp, maskpf.reshape(N_NODES, 1), mjf,
      w1a_s, b1_s, w1b_s, w2d_s, b2d_s, w3f_s, w3a_s, b3_s, w4_s, b4_s)

    weo, beo = p["gnn"]["embedding_out"]["w"], p["gnn"]["embedding_out"]["b"].reshape(1, -1)
    wad1, bad1 = lin("atom_decoder", "l1")
    wad2, bad2 = lin("atom_decoder", "l2")
    wrd1, brd1 = lin("residue_decoder", "l1")
    wrd2, brd2 = lin("residue_decoder", "l2")

    outa, outr = pl.pallas_call(
        _post_kernel,
        out_shape=(jax.ShapeDtypeStruct((N_ATOMS, N_DIMS + ATOM_NF), jnp.float32),
                   jax.ShapeDtypeStruct((N_RES, N_DIMS + RESIDUE_NF), jnp.float32)),
    )(featp, perm.reshape(N_NODES, 1), mask.reshape(N_NODES, 1),
      weo, beo, wad1, bad1, wad2, bad2, wrd1, brd1, wrd2, brd2)

    return outa, outr


def kernel(xh_atoms, xh_residues, xh_intersh, xh_intershp, t,
           mask_atoms, mask_residues, mask_intersh, mask_intershp, params):
    return _run(xh_atoms, xh_residues, t, mask_atoms, mask_residues, params)


# speedup vs baseline: 325.1173x; 1.2035x over previous
import jax
import jax.numpy as jnp
from jax.experimental import pallas as pl
from jax.experimental.pallas import tpu as pltpu

N_DIMS = 3
ATOM_NF = 16
RESIDUE_NF = 21
JOINT_NF = 16
HIDDEN_NF = 64
N_LAYERS = 4
NORM_FACTOR = 100.0
N_BATCH = 16
N_ATOMS = 320
N_RES = 1600
N_NODES = N_ATOMS + N_RES
TILE = 64
NT = N_NODES // TILE


def _silu(x):
    return 0.5 * x * (1.0 + jnp.tanh(0.5 * x))


def _perm_matrix(permcol):
    iota = jax.lax.broadcasted_iota(jnp.int32, (N_NODES, N_NODES), 1)
    return (permcol == iota).astype(jnp.float32)


def _pre_kernel(xa_ref, xr_ref, t_ref, perm_ref,
                wa1, ba1, wa2, ba2,
                wr1, br1, wr2, br2,
                we, be,
                out_ref):
    xa = xa_ref[...]
    xr = xr_ref[...]
    ha = xa[:, N_DIMS:]
    hr = xr[:, N_DIMS:]
    ha = _silu(ha @ wa1[...] + ba1[...]) @ wa2[...] + ba2[...]
    hr = _silu(hr @ wr1[...] + br1[...]) @ wr2[...] + br2[...]
    we_full = we[...]
    we_x = we_full[:N_DIMS, :]
    we_h = we_full[N_DIMS:N_DIMS + JOINT_NF, :]
    we_t = we_full[N_DIMS + JOINT_NF:, :]
    tval = t_ref[...]
    tcontrib = tval * we_t
    fa = xa[:, :N_DIMS] @ we_x + ha @ we_h + tcontrib + be[...]
    fr = xr[:, :N_DIMS] @ we_x + hr @ we_h + tcontrib + be[...]
    feat = jnp.concatenate([fa, fr], axis=0)
    out_ref[...] = _perm_matrix(perm_ref[...]) @ feat


TJ = 128
TJH = TJ // 2
NTJ = N_NODES // TJ
FOLD = 2 * HIDDEN_NF


def _gcl_kernel(bounds_ref,
                feat0_ref,
                mcol_ref,
                mjf_ref,
                w1a, b1, w1b, w2d, b2d,
                w3f, w3a, b3, w4, b4,
                out_ref,
                fbuf,
                a2buf,
                bfbuf,
                aggbuf):
    l = pl.program_id(0)
    ph = pl.program_id(1)

    cur = jax.lax.rem(l, 2)
    nxt = 1 - cur

    @pl.when(ph == 0)
    def _pre_step():
        @pl.when(l == 0)
        def _copy_in():
            fbuf[0] = feat0_ref[...]

        f = fbuf[cur]
        a = f @ w1a[0] + b1[0]
        a2buf[...] = jnp.concatenate([a, a], axis=1)
        b = f @ w1b[0]
        br = b.reshape(NTJ, 2, TJH, HIDDEN_NF)
        bfbuf[...] = jnp.concatenate([br[:, 0], br[:, 1]], axis=-1)

    @pl.when((ph >= 1) & (ph <= NT))
    def _row_tile():
        i = ph - 1
        ioff = pl.multiple_of(i * TILE, TILE)
        a2 = a2buf[pl.ds(ioff, TILE), :]
        mi3 = mcol_ref[pl.ds(ioff, TILE), :].reshape(TILE, 1, 1)
        jlo = bounds_ref[0, i]
        jhi = bounds_ref[1, i]

        def body(j, acc):
            bf = bfbuf[j]
            u = _silu(a2[:, None, :] + bf[None, :, :])
            m = _silu(u.reshape(TILE * TJH, FOLD) @ w2d[0] + b2d[0])
            m = m.reshape(TILE, TJH, FOLD)
            adjf = (mi3 == mjf_ref[j][None, :, :]).astype(jnp.float32)
            s = jnp.sum(m * adjf, axis=1)
            return acc + s[:, :HIDDEN_NF] + s[:, HIDDEN_NF:]

        acc = jax.lax.fori_loop(jlo, jhi + 1, body,
                                jnp.zeros((TILE, HIDDEN_NF), jnp.float32))
        aggbuf[pl.ds(ioff, TILE), :] = acc

    @pl.when(ph == NT + 1)
    def _node_step():
        f = fbuf[cur]
        agg = aggbuf[...] * (1.0 / NORM_FACTOR)
        tmp = _silu(f @ w3f[0] + agg @ w3a[0] + b3[0])
        newf = f + tmp @ w4[0] + b4[0]
        fbuf[nxt] = newf

        @pl.when(l == N_LAYERS - 1)
        def _emit():
            out_ref[...] = newf


def _post_kernel(featp_ref, perm_ref, maski_ref,
                 weo, beo,
                 wad1, bad1, wad2, bad2,
                 wrd1, brd1, wrd2, brd2,
                 outa_ref, outr_ref):
    pmat = _perm_matrix(perm_ref[...])
    feat = jax.lax.dot_general(pmat, featp_ref[...], (((0,), (0,)), ((), ())))
    out = feat @ weo[...] + beo[...]
    vel = out[:, :N_DIMS]
    hfin = out[:, N_DIMS:N_DIMS + JOINT_NF]
    ha = _silu(hfin[:N_ATOMS] @ wad1[...] + bad1[...]) @ wad2[...] + bad2[...]
    hr = _silu(hfin[N_ATOMS:] @ wrd1[...] + brd1[...]) @ wrd2[...] + brd2[...]
    maski = maski_ref[...]
    batches = jax.lax.broadcasted_iota(jnp.int32, (N_NODES, N_BATCH), 1)
    onehot = (maski == batches).astype(jnp.float32)
    seg = jax.lax.dot_general(onehot, vel, (((0,), (0,)), ((), ())))
    cnt = jnp.sum(onehot, axis=0, keepdims=True).T
    mean = seg / jnp.maximum(cnt, 1.0)
    vel = vel - onehot @ mean
    outa_ref[...] = jnp.concatenate([vel[:N_ATOMS], ha], axis=1)
    outr_ref[...] = jnp.concatenate([vel[N_ATOMS:], hr], axis=1)


@jax.jit
def _run(xh_atoms, xh_residues, t, mask_atoms, mask_residues, params):
    mask = jnp.concatenate([mask_atoms, mask_residues]).astype(jnp.int32)
    perm = jnp.argsort(mask, stable=True).astype(jnp.int32)
    maskp = mask[perm]
    maskpf = maskp.astype(jnp.float32)

    mt = maskp.reshape(NT, TILE)
    vmin = mt[:, 0]
    vmax = mt[:, -1]
    jlo = (jnp.searchsorted(maskp, vmin, side="left") // TJ).astype(jnp.int32)
    jhi = ((jnp.searchsorted(maskp, vmax, side="right") - 1) // TJ).astype(jnp.int32)
    bounds = jnp.stack([jlo, jhi])

    p = params

    def lin(name, sub=None):
        q = p[name] if sub is None else p[name][sub]
        return q["w"], q["b"].reshape(1, -1)

    wa1, ba1 = lin("atom_encoder", "l1")
    wa2, ba2 = lin("atom_encoder", "l2")
    wr1, br1 = lin("residue_encoder", "l1")
    wr2, br2 = lin("residue_encoder", "l2")
    we, be = p["gnn"]["embedding"]["w"], p["gnn"]["embedding"]["b"].reshape(1, -1)

    featp = pl.pallas_call(
        _pre_kernel,
        out_shape=jax.ShapeDtypeStruct((N_NODES, HIDDEN_NF), jnp.float32),
    )(xh_atoms, xh_residues, t.reshape(1, 1), perm.reshape(N_NODES, 1),
      wa1, ba1, wa2, ba2, wr1, br1, wr2, br2, we, be)

    gcl = p["gnn"]["gcl"]

    def stack(path):
        return jnp.stack([path(layer) for layer in gcl])

    def bdiag(w):
        return jnp.kron(jnp.eye(2, dtype=w.dtype), w)

    w1a_s = stack(lambda q: q["edge_mlp"]["l1"]["w"][:HIDDEN_NF, :])
    w1b_s = stack(lambda q: q["edge_mlp"]["l1"]["w"][HIDDEN_NF:, :])
    b1_s = stack(lambda q: q["edge_mlp"]["l1"]["b"].reshape(1, HIDDEN_NF))
    w2d_s = stack(lambda q: bdiag(q["edge_mlp"]["l2"]["w"]))
    b2d_s = stack(lambda q: jnp.tile(q["edge_mlp"]["l2"]["b"].reshape(1, HIDDEN_NF), (1, 2)))
    w3f_s = stack(lambda q: q["node_mlp"]["l1"]["w"][:HIDDEN_NF, :])
    w3a_s = stack(lambda q: q["node_mlp"]["l1"]["w"][HIDDEN_NF:, :])
    b3_s = stack(lambda q: q["node_mlp"]["l1"]["b"].reshape(1, HIDDEN_NF))
    w4_s = stack(lambda q: q["node_mlp"]["l2"]["w"])
    b4_s = stack(lambda q: q["node_mlp"]["l2"]["b"].reshape(1, HIDDEN_NF))

    def wspec(shape):
        nd = len(shape) - 1
        return pl.BlockSpec((1,) + shape[1:], lambda l, i: (l,) + (0,) * nd)

    mc = maskpf.reshape(NTJ, 2, TJH)
    mjf = jnp.concatenate(
        [jnp.broadcast_to(mc[:, 0, :, None], (NTJ, TJH, HIDDEN_NF)),
         jnp.broadcast_to(mc[:, 1, :, None], (NTJ, TJH, HIDDEN_NF))], axis=2)

    featp = pl.pallas_call(
        _gcl_kernel,
        grid=(N_LAYERS, NT + 2),
        in_specs=[
            pl.BlockSpec(memory_space=pltpu.SMEM),
            pl.BlockSpec((N_NODES, HIDDEN_NF), lambda l, i: (0, 0)),
            pl.BlockSpec((N_NODES, 1), lambda l, i: (0, 0)),
            pl.BlockSpec((NTJ, TJH, FOLD), lambda l, i: (0, 0, 0)),
            wspec(w1a_s.shape), wspec(b1_s.shape), wspec(w1b_s.shape),
            wspec(w2d_s.shape), wspec(b2d_s.shape),
            wspec(w3f_s.shape), wspec(w3a_s.shape), wspec(b3_s.shape),
            wspec(w4_s.shape), wspec(b4_s.shape),
        ],
        out_specs=pl.BlockSpec((N_NODES, HIDDEN_NF), lambda l, i: (0, 0)),
        out_shape=jax.ShapeDtypeStruct((N_NODES, HIDDEN_NF), jnp.float32),
        scratch_shapes=[pltpu.VMEM((2, N_NODES, HIDDEN_NF), jnp.float32),
                        pltpu.VMEM((N_NODES, FOLD), jnp.float32),
                        pltpu.VMEM((NTJ, TJH, FOLD), jnp.float32),
                        pltpu.VMEM((N_NODES, HIDDEN_NF), jnp.float32)],
    )(bounds, featp, maskpf.reshape(N_NODES, 1), mjf,
      w1a_s, b1_s, w1b_s, w2d_s, b2d_s, w3f_s, w3a_s, b3_s, w4_s, b4_s)

    weo, beo = p["gnn"]["embedding_out"]["w"], p["gnn"]["embedding_out"]["b"].reshape(1, -1)
    wad1, bad1 = lin("atom_decoder", "l1")
    wad2, bad2 = lin("atom_decoder", "l2")
    wrd1, brd1 = lin("residue_decoder", "l1")
    wrd2, brd2 = lin("residue_decoder", "l2")

    outa, outr = pl.pallas_call(
        _post_kernel,
        out_shape=(jax.ShapeDtypeStruct((N_ATOMS, N_DIMS + ATOM_NF), jnp.float32),
                   jax.ShapeDtypeStruct((N_RES, N_DIMS + RESIDUE_NF), jnp.float32)),
    )(featp, perm.reshape(N_NODES, 1), mask.reshape(N_NODES, 1),
      weo, beo, wad1, bad1, wad2, bad2, wrd1, brd1, wrd2, brd2)

    return outa, outr


def kernel(xh_atoms, xh_residues, xh_intersh, xh_intershp, t,
           mask_atoms, mask_residues, mask_intersh, mask_intershp, params):
    return _run(xh_atoms, xh_residues, t, mask_atoms, mask_residues, params)
